# baseline (speedup 1.0000x reference)
import numpy as np

import jax
import jax.numpy as jnp
from jax.experimental import pallas as pl
from jax.experimental.pallas import tpu as pltpu


def _band_selector(KW, Win, pad):
    kx = np.arange(KW).reshape(KW, 1, 1, 1)
    wi = np.arange(Win).reshape(1, Win, 1, 1)
    s = np.arange(2).reshape(1, 1, 2, 1)
    wp = np.arange(Win // 2).reshape(1, 1, 1, Win // 2)
    return (wi == 2 * wp + s + kx - pad).astype(np.float32)


_T1 = _band_selector(5, 32, 2)
_T2 = _band_selector(5, 16, 2)


def _band_mats(w_mat, sel, Cin, Cout):
    wr = w_mat.astype(jnp.float32).reshape(5, 5, Cin, Cout)
    b = jnp.einsum('xwsp,yxco->ywcspo', sel, wr)
    Win = sel.shape[1]
    return b.reshape(5, Win * Cin, Win * Cout).astype(jnp.bfloat16)


def _fused_kernel(xp_ref, B1_ref, b1t_ref, B2_ref, b2t_ref, wfc_ref,
                  fcb_ref, o_ref, h1p_ref):
    NB = xp_ref.shape[0]

    acc = None
    for ky in range(5):
        xs = xp_ref[:, ky:ky + 32, :].reshape(NB * 32, 96)
        d = jnp.dot(xs, B1_ref[ky], preferred_element_type=jnp.float32)
        acc = d if acc is None else acc + d
    zw = jnp.maximum(acc[:, :256], acc[:, 256:])
    zw = zw.reshape(NB * 16, 2, 256)
    z = jnp.maximum(zw[:, 0, :], zw[:, 1, :])
    h1 = jnp.maximum(z + b1t_ref[...], 0.0).astype(jnp.bfloat16)
    h1p_ref[:, 2:18, :] = h1.reshape(NB, 16, 256)
    h1p_ref[:, 0:2, :] = jnp.zeros((NB, 2, 256), jnp.bfloat16)
    h1p_ref[:, 18:20, :] = jnp.zeros((NB, 2, 256), jnp.bfloat16)

    acc2 = None
    for ky in range(5):
        xs = h1p_ref[:, ky:ky + 16, :].reshape(NB * 16, 256)
        d = jnp.dot(xs, B2_ref[ky], preferred_element_type=jnp.float32)
        acc2 = d if acc2 is None else acc2 + d
    zw2 = jnp.maximum(acc2[:, :256], acc2[:, 256:])
    zw2 = zw2.reshape(NB * 8, 2, 256)
    z2 = jnp.maximum(zw2[:, 0, :], zw2[:, 1, :])
    h2 = jnp.maximum(z2 + b2t_ref[...], 0.0).reshape(NB, 8, 256)

    logits = fcb_ref[...]
    for h in range(8):
        logits = logits + jnp.dot(h2[:, h, :], wfc_ref[h],
                                  preferred_element_type=jnp.float32)
    m = jnp.max(logits, axis=-1, keepdims=True)
    sh = logits - m
    lse = jnp.log(jnp.sum(jnp.exp(sh), axis=-1, keepdims=True))
    o_ref[...] = (sh - lse).astype(o_ref.dtype)


def kernel(w1, b1, w2, b2, fc_wT, fc_b, x_nchw):
    N = x_nchw.shape[0]
    NB = next(nb for nb in (128, 64, 32, 16, 8, 4, 2, 1) if N % nb == 0)

    x = jnp.transpose(x_nchw, (0, 2, 3, 1)).reshape(N, 32, 96)
    xp = jnp.pad(x, ((0, 0), (2, 2), (0, 0))).astype(jnp.bfloat16)
    B1 = _band_mats(w1, _T1, 3, 16)
    B2 = _band_mats(w2, _T2, 16, 32)
    b1t = jnp.tile(b1.astype(jnp.float32), (1, 16))
    b2t = jnp.tile(b2.astype(jnp.float32), (1, 8))
    wfc = fc_wT.astype(jnp.float32).reshape(8, 256, 10)
    fcb = fc_b.astype(jnp.float32).reshape(1, 10)

    return pl.pallas_call(
        _fused_kernel,
        out_shape=jax.ShapeDtypeStruct((N, 10), jnp.float32),
        grid_spec=pltpu.PrefetchScalarGridSpec(
            num_scalar_prefetch=0,
            grid=(N // NB,),
            in_specs=[
                pl.BlockSpec((NB, 36, 96), lambda i: (i, 0, 0)),
                pl.BlockSpec((5, 96, 512), lambda i: (0, 0, 0)),
                pl.BlockSpec((1, 256), lambda i: (0, 0)),
                pl.BlockSpec((5, 256, 512), lambda i: (0, 0, 0)),
                pl.BlockSpec((1, 256), lambda i: (0, 0)),
                pl.BlockSpec((8, 256, 10), lambda i: (0, 0, 0)),
                pl.BlockSpec((1, 10), lambda i: (0, 0)),
            ],
            out_specs=pl.BlockSpec((NB, 10), lambda i: (i, 0)),
            scratch_shapes=[pltpu.VMEM((NB, 20, 256), jnp.bfloat16)],
        ),
        compiler_params=pltpu.CompilerParams(
            dimension_semantics=("parallel",)),
    )(xp, B1, b1t, B2, b2t, wfc, fcb)


# speedup vs baseline: 1.3738x; 1.3738x over previous
import numpy as np

import jax
import jax.numpy as jnp
from jax.experimental import pallas as pl
from jax.experimental.pallas import tpu as pltpu


def _band_selector(KW, Win, pad):
    kx = np.arange(KW).reshape(KW, 1, 1, 1)
    wi = np.arange(Win).reshape(1, Win, 1, 1)
    s = np.arange(2).reshape(1, 1, 2, 1)
    wp = np.arange(Win // 2).reshape(1, 1, 1, Win // 2)
    return (wi == 2 * wp + s + kx - pad).astype(np.float32)


_T1 = _band_selector(5, 32, 2)
_T2 = _band_selector(5, 16, 2)


def _band_mats(w_mat, sel, Cin, Cout):
    wr = w_mat.astype(jnp.float32).reshape(5, 5, Cin, Cout)
    b = jnp.einsum('xwsp,yxco->ywcspo', sel, wr)
    Win = sel.shape[1]
    return b.reshape(5, Win * Cin, Win * Cout).astype(jnp.bfloat16)


def _fused_kernel(xp_ref, B1_ref, b1t_ref, B2_ref, b2t_ref, wfc_ref,
                  fcb_ref, o_ref, h1p_ref):
    NB = xp_ref.shape[1]

    acc = None
    for ky in range(5):
        xs = xp_ref[ky:ky + 32, :, :].reshape(32 * NB, 96)
        d = jnp.dot(xs, B1_ref[ky], preferred_element_type=jnp.float32)
        acc = d if acc is None else acc + d
    zw = jnp.maximum(acc[:, :256], acc[:, 256:])
    zw = zw.reshape(16, 2, NB, 256)
    z = jnp.maximum(zw[:, 0], zw[:, 1])
    h1 = jnp.maximum(z + b1t_ref[...], 0.0).astype(jnp.bfloat16)
    h1p_ref[2:18] = h1
    h1p_ref[0:2] = jnp.zeros((2, NB, 256), jnp.bfloat16)
    h1p_ref[18:20] = jnp.zeros((2, NB, 256), jnp.bfloat16)

    acc2 = None
    for ky in range(5):
        xs = h1p_ref[ky:ky + 16, :, :].reshape(16 * NB, 256)
        d = jnp.dot(xs, B2_ref[ky], preferred_element_type=jnp.float32)
        acc2 = d if acc2 is None else acc2 + d
    zw2 = jnp.maximum(acc2[:, :256], acc2[:, 256:])
    zw2 = zw2.reshape(8, 2, NB, 256)
    z2 = jnp.maximum(zw2[:, 0], zw2[:, 1])
    h2 = jnp.maximum(z2 + b2t_ref[...], 0.0)

    logits = fcb_ref[...]
    for h in range(8):
        logits = logits + jnp.dot(h2[h], wfc_ref[h],
                                  preferred_element_type=jnp.float32)
    m = jnp.max(logits, axis=-1, keepdims=True)
    sh = logits - m
    lse = jnp.log(jnp.sum(jnp.exp(sh), axis=-1, keepdims=True))
    o_ref[...] = (sh - lse).astype(o_ref.dtype)


def kernel(w1, b1, w2, b2, fc_wT, fc_b, x_nchw):
    N = x_nchw.shape[0]
    NB = next(nb for nb in (128, 64, 32, 16, 8, 4, 2, 1) if N % nb == 0)

    x = jnp.transpose(x_nchw, (2, 0, 3, 1)).reshape(32, N, 96)
    xp = jnp.pad(x, ((2, 2), (0, 0), (0, 0))).astype(jnp.bfloat16)
    B1 = _band_mats(w1, _T1, 3, 16)
    B2 = _band_mats(w2, _T2, 16, 32)
    b1t = jnp.tile(b1.astype(jnp.float32), (1, 16))
    b2t = jnp.tile(b2.astype(jnp.float32), (1, 8))
    wfc = fc_wT.astype(jnp.float32).reshape(8, 256, 10)
    fcb = fc_b.astype(jnp.float32).reshape(1, 10)

    return pl.pallas_call(
        _fused_kernel,
        out_shape=jax.ShapeDtypeStruct((N, 10), jnp.float32),
        grid_spec=pltpu.PrefetchScalarGridSpec(
            num_scalar_prefetch=0,
            grid=(N // NB,),
            in_specs=[
                pl.BlockSpec((36, NB, 96), lambda i: (0, i, 0)),
                pl.BlockSpec((5, 96, 512), lambda i: (0, 0, 0)),
                pl.BlockSpec((1, 256), lambda i: (0, 0)),
                pl.BlockSpec((5, 256, 512), lambda i: (0, 0, 0)),
                pl.BlockSpec((1, 256), lambda i: (0, 0)),
                pl.BlockSpec((8, 256, 10), lambda i: (0, 0, 0)),
                pl.BlockSpec((1, 10), lambda i: (0, 0)),
            ],
            out_specs=pl.BlockSpec((NB, 10), lambda i: (i, 0)),
            scratch_shapes=[pltpu.VMEM((20, NB, 256), jnp.bfloat16)],
        ),
        compiler_params=pltpu.CompilerParams(
            dimension_semantics=("parallel",)),
    )(xp, B1, b1t, B2, b2t, wfc, fcb)


# speedup vs baseline: 1.6166x; 1.1767x over previous
import numpy as np

import jax
import jax.numpy as jnp
from jax.experimental import pallas as pl
from jax.experimental.pallas import tpu as pltpu


def _band_selector(KW, Win, pad):
    kx = np.arange(KW).reshape(KW, 1, 1, 1)
    wi = np.arange(Win).reshape(1, Win, 1, 1)
    s = np.arange(2).reshape(1, 1, 2, 1)
    wp = np.arange(Win // 2).reshape(1, 1, 1, Win // 2)
    return (wi == 2 * wp + s + kx - pad).astype(np.float32)


_T1 = _band_selector(5, 32, 2)
_T2 = _band_selector(5, 16, 2)


def _band_mats(w_mat, sel, Cin, Cout):
    wr = w_mat.astype(jnp.float32).reshape(5, 5, Cin, Cout)
    b = jnp.einsum('xwsp,yxco->ywcspo', sel, wr)
    Win = sel.shape[1]
    return b.reshape(5, Win * Cin, Win * Cout).astype(jnp.bfloat16)


def _fused_kernel(xp_ref, B1_ref, b1t_ref, B2_ref, b2t_ref, wfc_ref,
                  fcb_ref, o_ref, xa_ref, xb_ref):
    NB = xp_ref.shape[1]

    for ky in range(5):
        xa_ref[:, 128 * ky:128 * ky + 96] = (
            xp_ref[ky:ky + 32, :, :].reshape(32 * NB, 96))
        xa_ref[:, 128 * ky + 96:128 * (ky + 1)] = (
            jnp.zeros((32 * NB, 32), jnp.bfloat16))
    acc = jnp.dot(xa_ref[...], B1_ref[...],
                  preferred_element_type=jnp.float32)
    zw = jnp.maximum(acc[:, :256], acc[:, 256:])
    zw = zw.reshape(16, 2, NB, 256)
    z = jnp.maximum(zw[:, 0], zw[:, 1])
    h1 = jnp.maximum(z + b1t_ref[...], 0.0).astype(jnp.bfloat16)

    for ky in range(5):
        lo, hi = max(0, 2 - ky), min(16, 18 - ky)
        xb_ref[lo * NB:hi * NB, 256 * ky:256 * (ky + 1)] = (
            h1[lo + ky - 2:hi + ky - 2].reshape((hi - lo) * NB, 256))
        if lo > 0:
            xb_ref[:lo * NB, 256 * ky:256 * (ky + 1)] = (
                jnp.zeros((lo * NB, 256), jnp.bfloat16))
        if hi < 16:
            xb_ref[hi * NB:, 256 * ky:256 * (ky + 1)] = (
                jnp.zeros(((16 - hi) * NB, 256), jnp.bfloat16))
    acc2 = jnp.dot(xb_ref[...], B2_ref[...],
                   preferred_element_type=jnp.float32)
    zw2 = jnp.maximum(acc2[:, :256], acc2[:, 256:])
    zw2 = zw2.reshape(8, 2, NB, 256)
    z2 = jnp.maximum(zw2[:, 0], zw2[:, 1])
    h2 = jnp.maximum(z2 + b2t_ref[...], 0.0)

    logits = fcb_ref[...]
    for h in range(8):
        logits = logits + jnp.dot(h2[h], wfc_ref[h],
                                  preferred_element_type=jnp.float32)
    m = jnp.max(logits, axis=-1, keepdims=True)
    sh = logits - m
    lse = jnp.log(jnp.sum(jnp.exp(sh), axis=-1, keepdims=True))
    o_ref[...] = (sh - lse).astype(o_ref.dtype)


def kernel(w1, b1, w2, b2, fc_wT, fc_b, x_nchw):
    N = x_nchw.shape[0]
    NB = next(nb for nb in (128, 64, 32, 16, 8, 4, 2, 1) if N % nb == 0)

    x = jnp.transpose(x_nchw, (2, 0, 3, 1)).reshape(32, N, 96)
    xp = jnp.pad(x, ((2, 2), (0, 0), (0, 0))).astype(jnp.bfloat16)
    B1 = _band_mats(w1, _T1, 3, 16)
    B1p = jnp.pad(B1, ((0, 0), (0, 32), (0, 0))).reshape(640, 512)
    B2p = _band_mats(w2, _T2, 16, 32).reshape(1280, 512)
    b1t = jnp.tile(b1.astype(jnp.float32), (1, 16))
    b2t = jnp.tile(b2.astype(jnp.float32), (1, 8))
    wfc = fc_wT.astype(jnp.float32).reshape(8, 256, 10)
    fcb = fc_b.astype(jnp.float32).reshape(1, 10)

    return pl.pallas_call(
        _fused_kernel,
        out_shape=jax.ShapeDtypeStruct((N, 10), jnp.float32),
        grid_spec=pltpu.PrefetchScalarGridSpec(
            num_scalar_prefetch=0,
            grid=(N // NB,),
            in_specs=[
                pl.BlockSpec((36, NB, 96), lambda i: (0, i, 0)),
                pl.BlockSpec((640, 512), lambda i: (0, 0)),
                pl.BlockSpec((1, 256), lambda i: (0, 0)),
                pl.BlockSpec((1280, 512), lambda i: (0, 0)),
                pl.BlockSpec((1, 256), lambda i: (0, 0)),
                pl.BlockSpec((8, 256, 10), lambda i: (0, 0, 0)),
                pl.BlockSpec((1, 10), lambda i: (0, 0)),
            ],
            out_specs=pl.BlockSpec((NB, 10), lambda i: (i, 0)),
            scratch_shapes=[pltpu.VMEM((32 * NB, 640), jnp.bfloat16),
                            pltpu.VMEM((16 * NB, 1280), jnp.bfloat16)],
        ),
        compiler_params=pltpu.CompilerParams(
            dimension_semantics=("parallel",)),
    )(xp, B1p, b1t, B2p, b2t, wfc, fcb)
